# baseline (speedup 1.0000x reference)
import jax
import jax.numpy as jnp
from jax import lax
from jax.experimental import pallas as pl
from jax.experimental.pallas import tpu as pltpu
from jax.experimental.pallas import tpu_sc as plsc

_N_ROWS = 100000
_DIM = 128
_SCALE = 1.0 / (128.0 ** 0.5)
_C = 400
_SUBS = (128, 128, 128, 16)
_NW = 32
_PW = 8
_WSPAN = _PW * _C
_WLAST = _N_ROWS - _WSPAN
_NBUF = 2
_NITER = _PW // _NBUF


def _scale_body(t_ref, o_ref):
    o_ref[...] = t_ref[...] * _SCALE


def _scaled_table(emb):
    return pl.pallas_call(
        _scale_body,
        out_shape=jax.ShapeDtypeStruct((_DIM, _DIM), jnp.float32),
    )(emb)


def _gather_body(table_hbm, idx_hbm, out_hbm, tbl_sh, idx_v, rows_v,
                 sg0, sg1, so0, so1):
    sg = (sg0, sg1)
    so = (so0, so1)
    s = lax.axis_index("s")
    w = s * 2 + lax.axis_index("c")
    base = jnp.minimum(w * _WSPAN, _WLAST)
    base = pl.multiple_of(base, 8)

    @pl.when(s == 0)
    def _():
        pltpu.sync_copy(table_hbm, rows_v.at[0, pl.ds(0, _DIM)])
        pltpu.sync_copy(rows_v.at[0, pl.ds(0, _DIM)], tbl_sh)

    pltpu.sync_copy(idx_hbm.at[pl.ds(base, _WSPAN)], idx_v)
    plsc.subcore_barrier()

    def sub_copies(i, b):
        ii = jnp.minimum(i, _PW - 1)
        coff = pl.multiple_of(ii * _C, 8)
        pairs = []
        sub_off = 0
        for n in _SUBS:
            sl = idx_v.at[pl.ds(coff + sub_off, n)]
            pairs.append((tbl_sh.at[sl], rows_v.at[b, pl.ds(sub_off, n)]))
            sub_off += n
        return pairs

    def start_gather(i, b):
        for src, dst in sub_copies(i, b):
            pltpu.async_copy(src, dst, sg[b])

    def wait_gather(i, b):
        for src, dst in sub_copies(i, b):
            pltpu.make_async_copy(src, dst, sg[b]).wait()

    def out_slice(i):
        ii = jnp.minimum(i, _PW - 1)
        off = pl.multiple_of(base + ii * _C, 8)
        return out_hbm.at[pl.ds(off, _C)]

    def start_wb(i, b):
        pltpu.async_copy(rows_v.at[b], out_slice(i), so[b])

    def wait_wb(i, b):
        pltpu.make_async_copy(rows_v.at[b], out_slice(i), so[b]).wait()

    for b in range(_NBUF):
        start_gather(jnp.int32(b), b)

    def wave(j, carry):
        for b in range(_NBUF):
            i = j * _NBUF + b
            wait_gather(i, b)
            start_wb(i, b)
        for b in range(_NBUF):
            i = j * _NBUF + b
            wait_wb(i, b)
            start_gather(i + _NBUF, b)
        return carry

    lax.fori_loop(0, _NITER - 1, wave, 0)

    for b in range(_NBUF):
        i = jnp.int32(_PW - _NBUF + b)
        wait_gather(i, b)
        start_wb(i, b)
    for b in range(_NBUF):
        i = jnp.int32(_PW - _NBUF + b)
        wait_wb(i, b)


def kernel(node_specie, embeddings):
    idx = node_specie.astype(jnp.int32)
    w = _scaled_table(embeddings)
    mesh = plsc.VectorSubcoreMesh(core_axis_name="c", subcore_axis_name="s")
    f = pl.kernel(
        _gather_body,
        mesh=mesh,
        out_type=jax.ShapeDtypeStruct((_N_ROWS, _DIM), jnp.float32),
        scratch_types=[
            pltpu.VMEM_SHARED((_DIM, _DIM), jnp.float32),
            pltpu.VMEM((_WSPAN,), jnp.int32),
            pltpu.VMEM((_NBUF, _C, _DIM), jnp.float32),
        ] + [pltpu.SemaphoreType.DMA] * (2 * _NBUF),
    )
    return f(w, idx)


# speedup vs baseline: 1.2260x; 1.2260x over previous
import jax
import jax.numpy as jnp
from jax import lax
from jax.experimental import pallas as pl
from jax.experimental.pallas import tpu as pltpu
from jax.experimental.pallas import tpu_sc as plsc

_N_ROWS = 100000
_DIM = 128
_SCALE = 1.0 / (128.0 ** 0.5)
_C = 80
_NW = 32
_PW = 40
_WSPAN = _PW * _C
_WLAST = _N_ROWS - _WSPAN
_NBUF = 8
_SKEW = _NBUF // 2


def _scale_body(t_ref, o_ref):
    o_ref[...] = t_ref[...] * _SCALE


def _scaled_table(emb):
    return pl.pallas_call(
        _scale_body,
        out_shape=jax.ShapeDtypeStruct((_DIM, _DIM), jnp.float32),
    )(emb)


def _gather_body(table_hbm, idx_hbm, out_hbm, tbl_sh, idx_v, rows_v, *sems):
    sg = sems[:_NBUF]
    so = sems[_NBUF:]
    s = lax.axis_index("s")
    w = s * 2 + lax.axis_index("c")
    base = jnp.minimum(w * _WSPAN, _WLAST)
    base = pl.multiple_of(base, 8)

    @pl.when(s == 0)
    def _():
        for h in range(2):
            pltpu.sync_copy(table_hbm.at[pl.ds(64 * h, 64)],
                            rows_v.at[0, pl.ds(0, 64)])
            pltpu.sync_copy(rows_v.at[0, pl.ds(0, 64)],
                            tbl_sh.at[pl.ds(64 * h, 64)])

    pltpu.sync_copy(idx_hbm.at[pl.ds(base, _WSPAN)], idx_v)
    plsc.subcore_barrier()

    def gather_pair(i, b):
        ii = jnp.minimum(i, _PW - 1)
        sl = idx_v.at[pl.ds(pl.multiple_of(ii * _C, 8), _C)]
        return tbl_sh.at[sl], rows_v.at[b]

    def start_gather(i, b):
        src, dst = gather_pair(i, b)
        pltpu.async_copy(src, dst, sg[b])

    def wait_gather(i, b):
        src, dst = gather_pair(i, b)
        pltpu.make_async_copy(src, dst, sg[b]).wait()

    def wb_pair(i, b):
        ii = jnp.minimum(i, _PW - 1)
        off = pl.multiple_of(base + ii * _C, 8)
        return rows_v.at[b], out_hbm.at[pl.ds(off, _C)]

    def start_wb(i, b):
        src, dst = wb_pair(i, b)
        pltpu.async_copy(src, dst, so[b])

    def wait_wb(i, b):
        src, dst = wb_pair(i, b)
        pltpu.make_async_copy(src, dst, so[b]).wait()

    def step(i, b, first):
        bn = (b + _SKEW) % _NBUF
        wait_gather(i, b)
        start_wb(i, b)
        if not first:
            wait_wb(i - _SKEW, bn)
        start_gather(i + _SKEW, bn)

    for b in range(_SKEW):
        start_gather(jnp.int32(b), b)
    for i in range(_SKEW):
        step(jnp.int32(i), i, True)

    def wave(j, carry):
        i0 = _SKEW + j * _NBUF
        for u in range(_NBUF):
            step(i0 + u, (_SKEW + u) % _NBUF, False)
        return carry

    lax.fori_loop(0, (_PW - 2 * _SKEW) // _NBUF, wave, 0)

    for u in range(_SKEW):
        i = _PW - _SKEW + u
        step(jnp.int32(i), i % _NBUF, False)

    for b in range(_SKEW):
        wait_gather(jnp.int32(_PW - 1), b)
    for u in range(_SKEW):
        i = _PW - _SKEW + u
        wait_wb(jnp.int32(i), i % _NBUF)


def kernel(node_specie, embeddings):
    idx = node_specie.astype(jnp.int32)
    w = _scaled_table(embeddings)
    mesh = plsc.VectorSubcoreMesh(core_axis_name="c", subcore_axis_name="s")
    f = pl.kernel(
        _gather_body,
        mesh=mesh,
        out_type=jax.ShapeDtypeStruct((_N_ROWS, _DIM), jnp.float32),
        scratch_types=[
            pltpu.VMEM_SHARED((_DIM, _DIM), jnp.float32),
            pltpu.VMEM((_WSPAN,), jnp.int32),
            pltpu.VMEM((_NBUF, _C, _DIM), jnp.float32),
        ] + [pltpu.SemaphoreType.DMA] * (2 * _NBUF),
    )
    return f(w, idx)
